# baseline (speedup 1.0000x reference)
import functools

import jax
import jax.numpy as jnp
from jax import lax
from jax.experimental import pallas as pl
from jax.experimental.pallas import tpu as pltpu
from jax.experimental.pallas import tpu_sc as plsc

N = 10000
E = 160000
IN_DIM = 256
GRAPH_DIM = 256
NUM_GRAPH = 4
NPAIR = NUM_GRAPH

BLK = 2048
NB = (N + BLK - 1) // BLK

NC, NS, L = 2, 16, 16
NW = NC * NS
EB = 128
EPT = 5120
LAST = NW - 1
EPT_LAST = E - LAST * EPT
GROUPS = EPT // L
GROUPS_LAST = EPT_LAST // L


def _rne_bf16_bits(u):
    return u + jnp.uint32(0x7FFF) + ((u >> 16) & jnp.uint32(1))


def _tc_pt_body(x_ref, wl_ref, bl_ref, ge_ref, go_ref, pt_ref):
    w2e = lax.dot_general(ge_ref[...], wl_ref[...], (((1,), (1,)), ((), ())),
                          preferred_element_type=jnp.float32)
    w2o = lax.dot_general(go_ref[...], wl_ref[...], (((1,), (1,)), ((), ())),
                          preferred_element_type=jnp.float32)
    ce = lax.dot_general(ge_ref[...], bl_ref[...], (((1,), (1,)), ((), ())),
                         preferred_element_type=jnp.float32)
    co = lax.dot_general(go_ref[...], bl_ref[...], (((1,), (1,)), ((), ())),
                         preferred_element_type=jnp.float32)
    pte = lax.dot_general(w2e, x_ref[...], (((1,), (1,)), ((), ())),
                          preferred_element_type=jnp.float32) + ce
    pto = lax.dot_general(w2o, x_ref[...], (((1,), (1,)), ((), ())),
                          preferred_element_type=jnp.float32) + co
    ue = lax.bitcast_convert_type(pte, jnp.uint32)
    uo = lax.bitcast_convert_type(pto, jnp.uint32)
    packed = ((_rne_bf16_bits(uo) & jnp.uint32(0xFFFF0000))
              | (_rne_bf16_bits(ue) >> 16))
    pt_ref[...] = lax.bitcast_convert_type(packed, jnp.int32)


def _tc_pt(x, W_lin, b_lin, ge, go):
    x = pltpu.with_memory_space_constraint(x, pltpu.MemorySpace.HBM)
    return pl.pallas_call(
        _tc_pt_body,
        grid=(NB,),
        in_specs=[
            pl.BlockSpec((BLK, IN_DIM), lambda i: (i, 0)),
            pl.BlockSpec((IN_DIM, GRAPH_DIM), lambda i: (0, 0)),
            pl.BlockSpec((1, GRAPH_DIM), lambda i: (0, 0)),
            pl.BlockSpec((NPAIR, GRAPH_DIM), lambda i: (0, 0)),
            pl.BlockSpec((NPAIR, GRAPH_DIM), lambda i: (0, 0)),
        ],
        out_specs=pl.BlockSpec((NPAIR, BLK), lambda i: (0, i)),
        out_shape=jax.ShapeDtypeStruct((NPAIR, N), jnp.int32),
    )(x, W_lin, b_lin, ge, go)


def _tc_h_body(x_ref, wl_ref, bl_ref, h_ref):
    h_ref[...] = (
        jnp.dot(x_ref[...], wl_ref[...], preferred_element_type=jnp.float32)
        + bl_ref[...])


def _tc_h(x, W_lin, b_lin):
    return pl.pallas_call(
        _tc_h_body,
        grid=(NB,),
        in_specs=[
            pl.BlockSpec((BLK, IN_DIM), lambda i: (i, 0)),
            pl.BlockSpec((IN_DIM, GRAPH_DIM), lambda i: (0, 0)),
            pl.BlockSpec((1, GRAPH_DIM), lambda i: (0, 0)),
        ],
        out_specs=pl.BlockSpec((BLK, GRAPH_DIM), lambda i: (i, 0)),
        out_shape=jax.ShapeDtypeStruct((N, GRAPH_DIM), jnp.float32),
    )(x, W_lin, b_lin)


def _sc_body(pt_hbm, edge_hbm, bias_hbm, out_hbm, pt_v, ev_v, out_v, bias_v):
    wid = lax.axis_index("s") * NC + lax.axis_index("c")
    base = wid * EPT
    pltpu.sync_copy(pt_hbm, pt_v)
    pltpu.sync_copy(edge_hbm.at[:, pl.ds(base, EPT_LAST)],
                    ev_v.at[:, pl.ds(0, EPT_LAST)])

    @pl.when(wid < LAST)
    def _():
        pltpu.sync_copy(edge_hbm.at[:, pl.ds(base + EPT_LAST, EPT - EPT_LAST)],
                        ev_v.at[:, pl.ds(EPT_LAST, EPT - EPT_LAST)])

    pltpu.sync_copy(bias_hbm, bias_v)

    biases = [bias_v[k, :] for k in range(NUM_GRAPH)]
    n_groups = jnp.where(wid == LAST, GROUPS_LAST, GROUPS)
    himask = jnp.int32(-65536)

    def lo(w):
        return plsc.bitcast(w << 16, jnp.float32)

    def hi(w):
        return plsc.bitcast(w & himask, jnp.float32)

    @plsc.parallel_loop(0, n_groups, unroll=4)
    def group(g):
        s = ev_v[0, pl.ds(g * L, L)]
        d = ev_v[1, pl.ds(g * L, L)]
        wd0 = plsc.load_gather(pt_v, [jnp.zeros((L,), jnp.int32), d])
        wd1 = plsc.load_gather(pt_v, [jnp.full((L,), 1, jnp.int32), d])
        ws0 = plsc.load_gather(pt_v, [jnp.full((L,), 2, jnp.int32), s])
        ws1 = plsc.load_gather(pt_v, [jnp.full((L,), 3, jnp.int32), s])
        ts = (lo(wd0) + lo(ws0) + biases[0],
              hi(wd0) + hi(ws0) + biases[1],
              lo(wd1) + lo(ws1) + biases[2],
              hi(wd1) + hi(ws1) + biases[3])
        obase = (g // 8) * (NUM_GRAPH * EB) + (g % 8) * L
        for k in range(NUM_GRAPH):
            out_v[pl.ds(obase + k * EB, L)] = 1.0 / (1.0 + jnp.exp(-ts[k]))

    head = EPT_LAST * NUM_GRAPH
    pltpu.sync_copy(out_v.at[pl.ds(0, head)],
                    out_hbm.at[pl.ds(base * NUM_GRAPH, head)])

    @pl.when(wid < LAST)
    def _():
        rest = (EPT - EPT_LAST) * NUM_GRAPH
        pltpu.sync_copy(out_v.at[pl.ds(head, rest)],
                        out_hbm.at[pl.ds(base * NUM_GRAPH + head, rest)])


@functools.partial(
    pl.kernel,
    mesh=plsc.VectorSubcoreMesh(core_axis_name="c", subcore_axis_name="s"),
    out_type=jax.ShapeDtypeStruct((E * NUM_GRAPH,), jnp.float32),
    compiler_params=pltpu.CompilerParams(needs_layout_passes=False),
    scratch_types=[
        pltpu.VMEM((NPAIR, N), jnp.int32),
        pltpu.VMEM((2, EPT), jnp.int32),
        pltpu.VMEM((EPT * NUM_GRAPH,), jnp.float32),
        pltpu.VMEM((NUM_GRAPH, L), jnp.float32),
    ],
)
def _sc_gate(pt_hbm, edge_hbm, bias_hbm, out_hbm, pt_v, ev_v, out_v, bias_v):
    _sc_body(pt_hbm, edge_hbm, bias_hbm, out_hbm, pt_v, ev_v, out_v, bias_v)


def kernel(x, edge_index, W_lin, b_lin, W_gate, b_gate):
    g8 = jnp.concatenate([W_gate[:, :GRAPH_DIM], W_gate[:, GRAPH_DIM:]], axis=0)
    ge = g8[0::2]
    go = g8[1::2]
    bl = b_lin.reshape(1, GRAPH_DIM)
    bias_b = jnp.broadcast_to(b_gate[:, None], (NUM_GRAPH, L))

    pt = _tc_pt(x, W_lin, bl, ge, go)
    out = _sc_gate(pt, edge_index, bias_b)
    h = _tc_h(x, W_lin, bl)
    factors = (out.reshape(E // EB, NUM_GRAPH, EB)
               .transpose(0, 2, 1)
               .reshape(E, NUM_GRAPH))
    return h, factors


# speedup vs baseline: 1.0115x; 1.0115x over previous
import functools

import jax
import jax.numpy as jnp
from jax import lax
from jax.experimental import pallas as pl
from jax.experimental.pallas import tpu as pltpu
from jax.experimental.pallas import tpu_sc as plsc

N = 10000
E = 160000
IN_DIM = 256
GRAPH_DIM = 256
NUM_GRAPH = 4
NPAIR = NUM_GRAPH

BLK = 2048
NB = (N + BLK - 1) // BLK

NC, NS, L = 2, 16, 16
NW = NC * NS
EB = 128
EPT = 5120
LAST = NW - 1
EPT_LAST = E - LAST * EPT
GROUPS = EPT // L
GROUPS_LAST = EPT_LAST // L


def _rne_bf16_bits(u):
    return u + jnp.uint32(0x7FFF) + ((u >> 16) & jnp.uint32(1))


def _tc_pt_body(x_ref, wl_ref, bl_ref, ge_ref, go_ref, pt_ref):
    w2e = lax.dot_general(ge_ref[...], wl_ref[...], (((1,), (1,)), ((), ())),
                          preferred_element_type=jnp.float32)
    w2o = lax.dot_general(go_ref[...], wl_ref[...], (((1,), (1,)), ((), ())),
                          preferred_element_type=jnp.float32)
    ce = lax.dot_general(ge_ref[...], bl_ref[...], (((1,), (1,)), ((), ())),
                         preferred_element_type=jnp.float32)
    co = lax.dot_general(go_ref[...], bl_ref[...], (((1,), (1,)), ((), ())),
                         preferred_element_type=jnp.float32)
    pte = lax.dot_general(w2e, x_ref[...], (((1,), (1,)), ((), ())),
                          preferred_element_type=jnp.float32) + ce
    pto = lax.dot_general(w2o, x_ref[...], (((1,), (1,)), ((), ())),
                          preferred_element_type=jnp.float32) + co
    ue = lax.bitcast_convert_type(pte, jnp.uint32)
    uo = lax.bitcast_convert_type(pto, jnp.uint32)
    packed = ((_rne_bf16_bits(uo) & jnp.uint32(0xFFFF0000))
              | (_rne_bf16_bits(ue) >> 16))
    pt_ref[...] = lax.bitcast_convert_type(packed, jnp.int32)


def _tc_pt(x, W_lin, b_lin, ge, go):
    x = pltpu.with_memory_space_constraint(x, pltpu.MemorySpace.HBM)
    return pl.pallas_call(
        _tc_pt_body,
        grid=(NB,),
        in_specs=[
            pl.BlockSpec((BLK, IN_DIM), lambda i: (i, 0)),
            pl.BlockSpec((IN_DIM, GRAPH_DIM), lambda i: (0, 0)),
            pl.BlockSpec((1, GRAPH_DIM), lambda i: (0, 0)),
            pl.BlockSpec((NPAIR, GRAPH_DIM), lambda i: (0, 0)),
            pl.BlockSpec((NPAIR, GRAPH_DIM), lambda i: (0, 0)),
        ],
        out_specs=pl.BlockSpec((NPAIR, BLK), lambda i: (0, i)),
        out_shape=jax.ShapeDtypeStruct((NPAIR, N), jnp.int32),
    )(x, W_lin, b_lin, ge, go)


def _tc_h_body(x_ref, wl_ref, bl_ref, h_ref):
    h_ref[...] = (
        jnp.dot(x_ref[...], wl_ref[...], preferred_element_type=jnp.float32)
        + bl_ref[...])


def _tc_h(x, W_lin, b_lin):
    return pl.pallas_call(
        _tc_h_body,
        grid=(NB,),
        in_specs=[
            pl.BlockSpec((BLK, IN_DIM), lambda i: (i, 0)),
            pl.BlockSpec((IN_DIM, GRAPH_DIM), lambda i: (0, 0)),
            pl.BlockSpec((1, GRAPH_DIM), lambda i: (0, 0)),
        ],
        out_specs=pl.BlockSpec((BLK, GRAPH_DIM), lambda i: (i, 0)),
        out_shape=jax.ShapeDtypeStruct((N, GRAPH_DIM), jnp.float32),
    )(x, W_lin, b_lin)


def _sc_body(pt_hbm, edge_hbm, bias_hbm, out_hbm, pt_v, ev_v, out_v, bias_v):
    wid = lax.axis_index("s") * NC + lax.axis_index("c")
    base = wid * EPT
    pltpu.sync_copy(pt_hbm, pt_v)
    pltpu.sync_copy(edge_hbm.at[:, pl.ds(base, EPT_LAST)],
                    ev_v.at[:, pl.ds(0, EPT_LAST)])

    @pl.when(wid < LAST)
    def _():
        pltpu.sync_copy(edge_hbm.at[:, pl.ds(base + EPT_LAST, EPT - EPT_LAST)],
                        ev_v.at[:, pl.ds(EPT_LAST, EPT - EPT_LAST)])

    pltpu.sync_copy(bias_hbm, bias_v)

    biases = [bias_v[k, :] for k in range(NUM_GRAPH)]
    n_groups = jnp.where(wid == LAST, GROUPS_LAST, GROUPS)
    himask = jnp.int32(-65536)

    def lo(w):
        return plsc.bitcast(w << 16, jnp.float32)

    def hi(w):
        return plsc.bitcast(w & himask, jnp.float32)

    @plsc.parallel_loop(0, n_groups, unroll=8)
    def group(g):
        s = ev_v[0, pl.ds(g * L, L)]
        d = ev_v[1, pl.ds(g * L, L)]
        wd0 = plsc.load_gather(pt_v, [jnp.zeros((L,), jnp.int32), d])
        wd1 = plsc.load_gather(pt_v, [jnp.full((L,), 1, jnp.int32), d])
        ws0 = plsc.load_gather(pt_v, [jnp.full((L,), 2, jnp.int32), s])
        ws1 = plsc.load_gather(pt_v, [jnp.full((L,), 3, jnp.int32), s])
        ts = (lo(wd0) + lo(ws0) + biases[0],
              hi(wd0) + hi(ws0) + biases[1],
              lo(wd1) + lo(ws1) + biases[2],
              hi(wd1) + hi(ws1) + biases[3])
        obase = (g // 8) * (NUM_GRAPH * EB) + (g % 8) * L
        for k in range(NUM_GRAPH):
            out_v[pl.ds(obase + k * EB, L)] = 1.0 / (1.0 + jnp.exp(-ts[k]))

    head = EPT_LAST * NUM_GRAPH
    pltpu.sync_copy(out_v.at[pl.ds(0, head)],
                    out_hbm.at[pl.ds(base * NUM_GRAPH, head)])

    @pl.when(wid < LAST)
    def _():
        rest = (EPT - EPT_LAST) * NUM_GRAPH
        pltpu.sync_copy(out_v.at[pl.ds(head, rest)],
                        out_hbm.at[pl.ds(base * NUM_GRAPH + head, rest)])


@functools.partial(
    pl.kernel,
    mesh=plsc.VectorSubcoreMesh(core_axis_name="c", subcore_axis_name="s"),
    out_type=jax.ShapeDtypeStruct((E * NUM_GRAPH,), jnp.float32),
    compiler_params=pltpu.CompilerParams(needs_layout_passes=False),
    scratch_types=[
        pltpu.VMEM((NPAIR, N), jnp.int32),
        pltpu.VMEM((2, EPT), jnp.int32),
        pltpu.VMEM((EPT * NUM_GRAPH,), jnp.float32),
        pltpu.VMEM((NUM_GRAPH, L), jnp.float32),
    ],
)
def _sc_gate(pt_hbm, edge_hbm, bias_hbm, out_hbm, pt_v, ev_v, out_v, bias_v):
    _sc_body(pt_hbm, edge_hbm, bias_hbm, out_hbm, pt_v, ev_v, out_v, bias_v)


def kernel(x, edge_index, W_lin, b_lin, W_gate, b_gate):
    g8 = jnp.concatenate([W_gate[:, :GRAPH_DIM], W_gate[:, GRAPH_DIM:]], axis=0)
    ge = g8[0::2]
    go = g8[1::2]
    bl = b_lin.reshape(1, GRAPH_DIM)
    bias_b = jnp.broadcast_to(b_gate[:, None], (NUM_GRAPH, L))

    pt = _tc_pt(x, W_lin, bl, ge, go)
    out = _sc_gate(pt, edge_index, bias_b)
    h = _tc_h(x, W_lin, bl)
    factors = (out.reshape(E // EB, NUM_GRAPH, EB)
               .transpose(0, 2, 1)
               .reshape(E, NUM_GRAPH))
    return h, factors


# speedup vs baseline: 1.0461x; 1.0342x over previous
import functools

import jax
import jax.numpy as jnp
from jax import lax
from jax.experimental import pallas as pl
from jax.experimental.pallas import tpu as pltpu
from jax.experimental.pallas import tpu_sc as plsc

N = 10000
E = 160000
IN_DIM = 256
GRAPH_DIM = 256
NUM_GRAPH = 4
NPAIR = NUM_GRAPH

BLK = 2048
NB = (N + BLK - 1) // BLK

NC, NS, L = 2, 16, 16
NW = NC * NS
EB = 128
EPT = 5120
LAST = NW - 1
EPT_LAST = E - LAST * EPT
GROUPS = EPT // L
GROUPS_LAST = EPT_LAST // L


def _rne_bf16_bits(u):
    return u + jnp.uint32(0x7FFF) + ((u >> 16) & jnp.uint32(1))


def _tc_pt_body(x_ref, wl_ref, bl_ref, wg_ref, pt_ref):
    wg = wg_ref[...]
    ge = jnp.concatenate(
        [wg[0:1, :GRAPH_DIM], wg[2:3, :GRAPH_DIM],
         wg[0:1, GRAPH_DIM:], wg[2:3, GRAPH_DIM:]], axis=0)
    go = jnp.concatenate(
        [wg[1:2, :GRAPH_DIM], wg[3:4, :GRAPH_DIM],
         wg[1:2, GRAPH_DIM:], wg[3:4, GRAPH_DIM:]], axis=0)
    w2e = lax.dot_general(ge, wl_ref[...], (((1,), (1,)), ((), ())),
                          preferred_element_type=jnp.float32)
    w2o = lax.dot_general(go, wl_ref[...], (((1,), (1,)), ((), ())),
                          preferred_element_type=jnp.float32)
    ce = lax.dot_general(ge, bl_ref[...], (((1,), (1,)), ((), ())),
                         preferred_element_type=jnp.float32)
    co = lax.dot_general(go, bl_ref[...], (((1,), (1,)), ((), ())),
                         preferred_element_type=jnp.float32)
    pte = lax.dot_general(w2e, x_ref[...], (((1,), (1,)), ((), ())),
                          preferred_element_type=jnp.float32) + ce
    pto = lax.dot_general(w2o, x_ref[...], (((1,), (1,)), ((), ())),
                          preferred_element_type=jnp.float32) + co
    ue = lax.bitcast_convert_type(pte, jnp.uint32)
    uo = lax.bitcast_convert_type(pto, jnp.uint32)
    packed = ((_rne_bf16_bits(uo) & jnp.uint32(0xFFFF0000))
              | (_rne_bf16_bits(ue) >> 16))
    pt_ref[...] = lax.bitcast_convert_type(packed, jnp.int32)


def _tc_pt(x, W_lin, b_lin, W_gate):
    x = pltpu.with_memory_space_constraint(x, pltpu.MemorySpace.HBM)
    return pl.pallas_call(
        _tc_pt_body,
        grid=(NB,),
        in_specs=[
            pl.BlockSpec((BLK, IN_DIM), lambda i: (i, 0)),
            pl.BlockSpec((IN_DIM, GRAPH_DIM), lambda i: (0, 0)),
            pl.BlockSpec((1, GRAPH_DIM), lambda i: (0, 0)),
            pl.BlockSpec((NUM_GRAPH, 2 * GRAPH_DIM), lambda i: (0, 0)),
        ],
        out_specs=pl.BlockSpec((NPAIR, BLK), lambda i: (0, i)),
        out_shape=jax.ShapeDtypeStruct((NPAIR, N), jnp.int32),
    )(x, W_lin, b_lin, W_gate)


def _tc_h_body(x_ref, wl_ref, bl_ref, h_ref):
    h_ref[...] = (
        jnp.dot(x_ref[...], wl_ref[...], preferred_element_type=jnp.float32)
        + bl_ref[...])


def _tc_h(x, W_lin, b_lin):
    return pl.pallas_call(
        _tc_h_body,
        grid=(NB,),
        in_specs=[
            pl.BlockSpec((BLK, IN_DIM), lambda i: (i, 0)),
            pl.BlockSpec((IN_DIM, GRAPH_DIM), lambda i: (0, 0)),
            pl.BlockSpec((1, GRAPH_DIM), lambda i: (0, 0)),
        ],
        out_specs=pl.BlockSpec((BLK, GRAPH_DIM), lambda i: (i, 0)),
        out_shape=jax.ShapeDtypeStruct((N, GRAPH_DIM), jnp.float32),
    )(x, W_lin, b_lin)


def _sc_body(pt_hbm, edge_hbm, bias_hbm, out_hbm, pt_v, ev_v, out_v, bias_v):
    wid = lax.axis_index("s") * NC + lax.axis_index("c")
    base = wid * EPT
    pltpu.sync_copy(pt_hbm, pt_v)
    pltpu.sync_copy(edge_hbm.at[:, pl.ds(base, EPT_LAST)],
                    ev_v.at[:, pl.ds(0, EPT_LAST)])
    tail_src = jnp.where(wid < LAST, base + EPT_LAST, 0)
    pltpu.sync_copy(edge_hbm.at[:, pl.ds(tail_src, EPT - EPT_LAST)],
                    ev_v.at[:, pl.ds(EPT_LAST, EPT - EPT_LAST)])

    pltpu.sync_copy(bias_hbm, bias_v)
    biases = [bias_v[k, :] for k in range(NUM_GRAPH)]
    himask = jnp.int32(-65536)

    def lo(w):
        return plsc.bitcast(w << 16, jnp.float32)

    def hi(w):
        return plsc.bitcast(w & himask, jnp.float32)

    @plsc.parallel_loop(0, GROUPS, unroll=8)
    def group(g):
        s = ev_v[0, pl.ds(g * L, L)]
        d = ev_v[1, pl.ds(g * L, L)]
        wd0 = plsc.load_gather(pt_v, [jnp.zeros((L,), jnp.int32), d])
        wd1 = plsc.load_gather(pt_v, [jnp.full((L,), 1, jnp.int32), d])
        ws0 = plsc.load_gather(pt_v, [jnp.full((L,), 2, jnp.int32), s])
        ws1 = plsc.load_gather(pt_v, [jnp.full((L,), 3, jnp.int32), s])
        ts = (lo(wd0) + lo(ws0) + biases[0],
              hi(wd0) + hi(ws0) + biases[1],
              lo(wd1) + lo(ws1) + biases[2],
              hi(wd1) + hi(ws1) + biases[3])
        obase = (g // 8) * (NUM_GRAPH * EB) + (g % 8) * L
        for k in range(NUM_GRAPH):
            out_v[pl.ds(obase + k * EB, L)] = 1.0 / (1.0 + jnp.exp(-ts[k]))

    head = EPT_LAST * NUM_GRAPH
    pltpu.sync_copy(out_v.at[pl.ds(0, head)],
                    out_hbm.at[pl.ds(base * NUM_GRAPH, head)])

    @pl.when(wid < LAST)
    def _():
        rest = (EPT - EPT_LAST) * NUM_GRAPH
        pltpu.sync_copy(out_v.at[pl.ds(head, rest)],
                        out_hbm.at[pl.ds(base * NUM_GRAPH + head, rest)])


@functools.partial(
    pl.kernel,
    mesh=plsc.VectorSubcoreMesh(core_axis_name="c", subcore_axis_name="s"),
    out_type=jax.ShapeDtypeStruct((E * NUM_GRAPH,), jnp.float32),
    compiler_params=pltpu.CompilerParams(needs_layout_passes=False),
    scratch_types=[
        pltpu.VMEM((NPAIR, N), jnp.int32),
        pltpu.VMEM((2, EPT), jnp.int32),
        pltpu.VMEM((EPT * NUM_GRAPH,), jnp.float32),
        pltpu.VMEM((NUM_GRAPH, L), jnp.float32),
    ],
)
def _sc_gate(pt_hbm, edge_hbm, bias_hbm, out_hbm, pt_v, ev_v, out_v, bias_v):
    _sc_body(pt_hbm, edge_hbm, bias_hbm, out_hbm, pt_v, ev_v, out_v, bias_v)


def kernel(x, edge_index, W_lin, b_lin, W_gate, b_gate):
    bl = b_lin.reshape(1, GRAPH_DIM)
    bias_b = jnp.broadcast_to(b_gate[:, None], (NUM_GRAPH, L))

    pt = _tc_pt(x, W_lin, bl, W_gate)
    out = _sc_gate(pt, edge_index, bias_b)
    h = _tc_h(x, W_lin, bl)
    factors = (out.reshape(E // EB, NUM_GRAPH, EB)
               .transpose(0, 2, 1)
               .reshape(E, NUM_GRAPH))
    return h, factors
